# baseline (speedup 1.0000x reference)
import jax
import jax.numpy as jnp
from jax.experimental import pallas as pl
from jax.experimental.pallas import tpu as pltpu

H = 2048
I = 1024
E = 16
G = 4
GS = 4
SH_I = 2048
T = 2048
IB = 256

NEG = -1e9


def _nt_dot(a, b):
    return jax.lax.dot_general(
        a, b, (((1,), (1,)), ((), ())), preferred_element_type=jnp.float32
    )


def _routing_kernel(hs_ref, gw_ref, sel_ref, hsb_ref):
    tb = hs_ref.shape[0]
    hs = hs_ref[...]
    hsb_ref[...] = hs.astype(jnp.bfloat16)
    logits = _nt_dot(hs, gw_ref[...])
    m = jnp.max(logits, axis=1, keepdims=True)
    ex = jnp.exp(logits - m)
    p = ex / jnp.sum(ex, axis=1, keepdims=True)

    idx = jax.lax.broadcasted_iota(jnp.int32, (tb, E), 1)
    m1 = jnp.max(p, axis=1, keepdims=True)
    i1 = jnp.min(jnp.where(p == m1, idx, E), axis=1, keepdims=True)
    mask1 = idx == i1
    p2 = jnp.where(mask1, -jnp.inf, p)
    m2 = jnp.max(p2, axis=1, keepdims=True)
    i2 = jnp.min(jnp.where(p2 == m2, idx, E), axis=1, keepdims=True)
    sel = mask1 | (idx == i2)

    pad = jnp.zeros((tb, E - 1), jnp.float32)
    for g in range(G):
        gm = (idx // GS) == g
        flat = jnp.where(gm & sel, p, 0.0)
        scal = jnp.sum(flat, axis=1, keepdims=True)
        fm = jnp.where(gm & sel, p, NEG)
        fmx = jnp.max(jnp.where(gm, fm, -jnp.inf), axis=1, keepdims=True)
        e = jnp.where(gm, jnp.exp(fm - fmx), 0.0)
        sm = e / jnp.sum(e, axis=1, keepdims=True)
        sel_ref[g, :, :] = jnp.concatenate((sm, scal, pad), axis=1)


def _routed_kernel(hs_ref, sel_ref, wmf_ref, wg_ref, wu_ref, wd_ref,
                   out_ref, xs_ref):
    j = pl.program_id(0)

    @pl.when(j == 0)
    def _():
        out_ref[...] = jnp.zeros_like(out_ref)

    @pl.when(j % 4 == 0)
    def _():
        sm = sel_ref[0, :, :E].astype(jnp.bfloat16)
        corr = _nt_dot(sm, wmf_ref[...].astype(jnp.bfloat16))
        x = hs_ref[...].astype(jnp.float32) + corr
        xs_ref[...] = x.astype(jnp.bfloat16)

    wg = wg_ref[0].astype(jnp.bfloat16)
    wu = wu_ref[0].astype(jnp.bfloat16)
    wd = wd_ref[0].astype(jnp.bfloat16)
    for c in range(4):
        rows = pl.ds(c * (T // 4), T // 4)
        x = xs_ref[rows, :]
        a = _nt_dot(x, wg)
        b = _nt_dot(x, wu)
        h = (jax.nn.silu(a) * b) * sel_ref[0, rows, E:E + 1]
        out_ref[rows, :] += _nt_dot(h.astype(jnp.bfloat16), wd)


def _shared_kernel(hs_ref, y_ref, wsg_ref, wsu_ref, wsd_ref, out_ref):
    s = pl.program_id(0)

    @pl.when(s == 0)
    def _():
        out_ref[...] = y_ref[...]

    wsg = wsg_ref[...].astype(jnp.bfloat16)
    wsu = wsu_ref[...].astype(jnp.bfloat16)
    wsd = wsd_ref[...].astype(jnp.bfloat16)
    for c in range(4):
        rows = pl.ds(c * (T // 4), T // 4)
        x = hs_ref[rows, :]
        a = _nt_dot(x, wsg)
        b = _nt_dot(x, wsu)
        h = jax.nn.silu(a) * b
        out_ref[rows, :] += _nt_dot(h.astype(jnp.bfloat16), wsd)


@jax.jit
def kernel(hidden_states, gate_w, Wg, Wu, Wd, Wm, Wsg, Wsu, Wsd):
    orig_shape = hidden_states.shape
    hs = hidden_states.reshape(T, H)

    wm_flat = jnp.transpose(Wm, (1, 0, 2)).reshape(H, G * GS)

    RTB = 512
    sel, hs_bf = pl.pallas_call(
        _routing_kernel,
        grid=(T // RTB,),
        in_specs=[
            pl.BlockSpec((RTB, H), lambda i: (i, 0)),
            pl.BlockSpec((E, H), lambda i: (0, 0)),
        ],
        out_specs=[
            pl.BlockSpec((G, RTB, 2 * E), lambda i: (0, i, 0)),
            pl.BlockSpec((RTB, H), lambda i: (i, 0)),
        ],
        out_shape=[
            jax.ShapeDtypeStruct((G, T, 2 * E), jnp.float32),
            jax.ShapeDtypeStruct((T, H), jnp.bfloat16),
        ],
        compiler_params=pltpu.CompilerParams(
            dimension_semantics=("arbitrary",),
        ),
    )(hs, gate_w)

    n_ib = I // IB
    y = pl.pallas_call(
        _routed_kernel,
        grid=(G * n_ib,),
        in_specs=[
            pl.BlockSpec((T, H), lambda j: (0, 0)),
            pl.BlockSpec((1, T, 2 * E), lambda j: (j // n_ib, 0, 0)),
            pl.BlockSpec((H, G * GS), lambda j: (0, 0)),
            pl.BlockSpec((1, IB, H), lambda j: (j // n_ib, j % n_ib, 0)),
            pl.BlockSpec((1, IB, H), lambda j: (j // n_ib, j % n_ib, 0)),
            pl.BlockSpec((1, H, IB), lambda j: (j // n_ib, 0, j % n_ib)),
        ],
        out_specs=pl.BlockSpec((T, H), lambda j: (0, 0)),
        out_shape=jax.ShapeDtypeStruct((T, H), jnp.float32),
        scratch_shapes=[pltpu.VMEM((T, H), jnp.bfloat16)],
        compiler_params=pltpu.CompilerParams(
            dimension_semantics=("arbitrary",),
        ),
    )(hs_bf, sel, wm_flat, Wg, Wu, Wd)

    out = pl.pallas_call(
        _shared_kernel,
        grid=(SH_I // IB,),
        in_specs=[
            pl.BlockSpec((T, H), lambda s: (0, 0)),
            pl.BlockSpec((T, H), lambda s: (0, 0)),
            pl.BlockSpec((IB, H), lambda s: (s, 0)),
            pl.BlockSpec((IB, H), lambda s: (s, 0)),
            pl.BlockSpec((H, IB), lambda s: (0, s)),
        ],
        out_specs=pl.BlockSpec((T, H), lambda s: (0, 0)),
        out_shape=jax.ShapeDtypeStruct((T, H), jnp.float32),
        compiler_params=pltpu.CompilerParams(
            dimension_semantics=("arbitrary",),
        ),
    )(hs_bf, y, Wsg, Wsu, Wsd)

    return out.reshape(orig_shape)


# speedup vs baseline: 1.0041x; 1.0041x over previous
import jax
import jax.numpy as jnp
from jax.experimental import pallas as pl
from jax.experimental.pallas import tpu as pltpu

H = 2048
I = 1024
E = 16
G = 4
GS = 4
SH_I = 2048
T = 2048
IB = 256

NEG = -1e9


def _nt_dot(a, b):
    return jax.lax.dot_general(
        a, b, (((1,), (1,)), ((), ())), preferred_element_type=jnp.float32
    )


def _routing_kernel(hs_ref, gw_ref, sel_ref, hsb_ref):
    tb = hs_ref.shape[0]
    hs = hs_ref[...]
    hsb_ref[...] = hs.astype(jnp.bfloat16)
    logits = _nt_dot(hs, gw_ref[...])
    m = jnp.max(logits, axis=1, keepdims=True)
    ex = jnp.exp(logits - m)
    p = ex / jnp.sum(ex, axis=1, keepdims=True)

    idx = jax.lax.broadcasted_iota(jnp.int32, (tb, E), 1)
    m1 = jnp.max(p, axis=1, keepdims=True)
    i1 = jnp.min(jnp.where(p == m1, idx, E), axis=1, keepdims=True)
    mask1 = idx == i1
    p2 = jnp.where(mask1, -jnp.inf, p)
    m2 = jnp.max(p2, axis=1, keepdims=True)
    i2 = jnp.min(jnp.where(p2 == m2, idx, E), axis=1, keepdims=True)
    sel = mask1 | (idx == i2)

    pad = jnp.zeros((tb, E - 1), jnp.float32)
    for g in range(G):
        gm = (idx // GS) == g
        flat = jnp.where(gm & sel, p, 0.0)
        scal = jnp.sum(flat, axis=1, keepdims=True)
        fm = jnp.where(gm & sel, p, NEG)
        fmx = jnp.max(jnp.where(gm, fm, -jnp.inf), axis=1, keepdims=True)
        e = jnp.where(gm, jnp.exp(fm - fmx), 0.0)
        sm = e / jnp.sum(e, axis=1, keepdims=True)
        sel_ref[g, :, :] = jnp.concatenate((sm, scal, pad), axis=1)


def _routed_kernel(hs_ref, sel_ref, wmf_ref, wg_ref, wu_ref, wd_ref,
                   out_ref, xs_ref):
    j = pl.program_id(0)

    @pl.when(j == 0)
    def _():
        out_ref[...] = jnp.zeros_like(out_ref)

    @pl.when(j % 4 == 0)
    def _():
        sm = sel_ref[0, :, :E].astype(jnp.bfloat16)
        corr = _nt_dot(sm, wmf_ref[...].astype(jnp.bfloat16))
        x = hs_ref[...].astype(jnp.float32) + corr
        xs_ref[...] = x.astype(jnp.bfloat16)

    wg = wg_ref[0].astype(jnp.bfloat16)
    wu = wu_ref[0].astype(jnp.bfloat16)
    wd = wd_ref[0].astype(jnp.bfloat16)
    for c in range(2):
        rows = pl.ds(c * (T // 2), T // 2)
        x = xs_ref[rows, :]
        a = _nt_dot(x, wg)
        b = _nt_dot(x, wu)
        h = (jax.nn.silu(a) * b) * sel_ref[0, rows, E:E + 1]
        out_ref[rows, :] += _nt_dot(h.astype(jnp.bfloat16), wd)


def _shared_kernel(hs_ref, y_ref, wsg_ref, wsu_ref, wsd_ref, out_ref):
    s = pl.program_id(0)

    @pl.when(s == 0)
    def _():
        out_ref[...] = y_ref[...]

    wsg = wsg_ref[...].astype(jnp.bfloat16)
    wsu = wsu_ref[...].astype(jnp.bfloat16)
    wsd = wsd_ref[...].astype(jnp.bfloat16)
    for c in range(2):
        rows = pl.ds(c * (T // 2), T // 2)
        x = hs_ref[rows, :]
        a = _nt_dot(x, wsg)
        b = _nt_dot(x, wsu)
        h = jax.nn.silu(a) * b
        out_ref[rows, :] += _nt_dot(h.astype(jnp.bfloat16), wsd)


@jax.jit
def kernel(hidden_states, gate_w, Wg, Wu, Wd, Wm, Wsg, Wsu, Wsd):
    orig_shape = hidden_states.shape
    hs = hidden_states.reshape(T, H)

    wm_flat = jnp.transpose(Wm, (1, 0, 2)).reshape(H, G * GS)

    RTB = 512
    sel, hs_bf = pl.pallas_call(
        _routing_kernel,
        grid=(T // RTB,),
        in_specs=[
            pl.BlockSpec((RTB, H), lambda i: (i, 0)),
            pl.BlockSpec((E, H), lambda i: (0, 0)),
        ],
        out_specs=[
            pl.BlockSpec((G, RTB, 2 * E), lambda i: (0, i, 0)),
            pl.BlockSpec((RTB, H), lambda i: (i, 0)),
        ],
        out_shape=[
            jax.ShapeDtypeStruct((G, T, 2 * E), jnp.float32),
            jax.ShapeDtypeStruct((T, H), jnp.bfloat16),
        ],
        compiler_params=pltpu.CompilerParams(
            dimension_semantics=("arbitrary",),
        ),
    )(hs, gate_w)

    n_ib = I // IB
    y = pl.pallas_call(
        _routed_kernel,
        grid=(G * n_ib,),
        in_specs=[
            pl.BlockSpec((T, H), lambda j: (0, 0)),
            pl.BlockSpec((1, T, 2 * E), lambda j: (j // n_ib, 0, 0)),
            pl.BlockSpec((H, G * GS), lambda j: (0, 0)),
            pl.BlockSpec((1, IB, H), lambda j: (j // n_ib, j % n_ib, 0)),
            pl.BlockSpec((1, IB, H), lambda j: (j // n_ib, j % n_ib, 0)),
            pl.BlockSpec((1, H, IB), lambda j: (j // n_ib, 0, j % n_ib)),
        ],
        out_specs=pl.BlockSpec((T, H), lambda j: (0, 0)),
        out_shape=jax.ShapeDtypeStruct((T, H), jnp.float32),
        scratch_shapes=[pltpu.VMEM((T, H), jnp.bfloat16)],
        compiler_params=pltpu.CompilerParams(
            dimension_semantics=("arbitrary",),
        ),
    )(hs_bf, sel, wm_flat, Wg, Wu, Wd)

    out = pl.pallas_call(
        _shared_kernel,
        grid=(SH_I // IB,),
        in_specs=[
            pl.BlockSpec((T, H), lambda s: (0, 0)),
            pl.BlockSpec((T, H), lambda s: (0, 0)),
            pl.BlockSpec((IB, H), lambda s: (s, 0)),
            pl.BlockSpec((IB, H), lambda s: (s, 0)),
            pl.BlockSpec((H, IB), lambda s: (0, s)),
        ],
        out_specs=pl.BlockSpec((T, H), lambda s: (0, 0)),
        out_shape=jax.ShapeDtypeStruct((T, H), jnp.float32),
        compiler_params=pltpu.CompilerParams(
            dimension_semantics=("arbitrary",),
        ),
    )(hs_bf, y, Wsg, Wsu, Wsd)

    return out.reshape(orig_shape)
